# baseline (speedup 1.0000x reference)
import functools

import jax
import jax.numpy as jnp
from jax import lax
from jax.experimental import pallas as pl
from jax.experimental.pallas import tpu as pltpu
from jax.experimental.pallas import tpu_sc as plsc

VOCAB = 1000000
EMBED = 64
MAXSEQ = 200
BATCH = 4096
SEQ = 200

TOTAL_ROWS = BATCH * SEQ
LANES = 16
VPR = EMBED // LANES
PAIR = 2 * EMBED

_INFO = plsc.get_sparse_core_info()
NC = _INFO.num_cores
NS = _INFO.num_subcores
NW = NC * NS
ROWS_PER_W = TOTAL_ROWS // NW
CHUNK = 128
NCHUNK = ROWS_PER_W // CHUNK


def _rsqrt(x):
    i = lax.bitcast_convert_type(x, jnp.int32)
    i = jnp.int32(0x5F3759DF) - lax.shift_right_logical(i, 1)
    y = lax.bitcast_convert_type(i, jnp.float32)
    for _ in range(3):
        y = y * (jnp.float32(1.5) - jnp.float32(0.5) * x * y * y)
    return y


def _make_sc_call():
    mesh = plsc.VectorSubcoreMesh(core_axis_name="c", subcore_axis_name="s")

    @functools.partial(
        pl.kernel,
        mesh=mesh,
        compiler_params=pltpu.CompilerParams(
            needs_layout_passes=False, use_tc_tiling_on_sc=True),
        out_type=jax.ShapeDtypeStruct((TOTAL_ROWS * EMBED,), jnp.float32),
        scratch_types=[
            pltpu.VMEM((NCHUNK, CHUNK), jnp.int32),
            pltpu.VMEM((NCHUNK, CHUNK), jnp.int32),
            pltpu.VMEM((2, CHUNK, PAIR), jnp.float32),
            pltpu.VMEM((2, CHUNK * EMBED), jnp.float32),
            pltpu.VMEM((MAXSEQ * EMBED,), jnp.float32),
            pltpu.VMEM((2 * EMBED,), jnp.float32),
            pltpu.SemaphoreType.DMA,
            pltpu.SemaphoreType.DMA,
            pltpu.SemaphoreType.DMA,
            pltpu.SemaphoreType.DMA,
        ],
    )
    def sc_embed(ids_hbm, tblp_hbm, pos_hbm, gb_hbm, out_hbm,
                 idx_all, pidx_all, rows2, ost2, pos_v, gb_v,
                 gsem0, gsem1, osem0, osem1):
        wid = lax.axis_index("s") * NC + lax.axis_index("c")
        wstart = wid * ROWS_PER_W
        gsems = (gsem0, gsem1)
        osems = (osem0, osem1)

        pltpu.sync_copy(ids_hbm.at[wid], idx_all)
        pltpu.sync_copy(pos_hbm, pos_v)
        pltpu.sync_copy(gb_hbm, gb_v)

        @plsc.parallel_loop(0, NCHUNK * (CHUNK // LANES), unroll=8)
        def _mk(i):
            c = lax.shift_right_logical(i, 3)
            col = (i & 7) * LANES
            v = idx_all[c, pl.ds(col, LANES)]
            pidx_all[c, pl.ds(col, LANES)] = lax.shift_right_logical(v, 1)

        inv_n = jnp.float32(1.0 / EMBED)
        eps = jnp.float32(1e-5)

        def fire_gather(c, b):
            pltpu.async_copy(tblp_hbm.at[pidx_all.at[c]], rows2.at[b],
                             gsems[b])

        def wait_gather(c, b):
            pltpu.make_async_copy(tblp_hbm.at[pidx_all.at[c]], rows2.at[b],
                                  gsems[b]).wait()

        def fire_out(base, b):
            pltpu.async_copy(ost2.at[b],
                             out_hbm.at[pl.ds(base * EMBED, CHUNK * EMBED)],
                             osems[b])

        def wait_out(base, b):
            pltpu.make_async_copy(ost2.at[b],
                                  out_hbm.at[pl.ds(base * EMBED, CHUNK * EMBED)],
                                  osems[b]).wait()

        def process(c, b):
            base = wstart + c * CHUNK
            wait_gather(c, b)

            @pl.when(c >= 2)
            def _():
                wait_out(base, b)

            @plsc.parallel_loop(0, CHUNK // LANES, unroll=2)
            def _grp(gidx):
                raws = idx_all[c, pl.ds(gidx * LANES, LANES)]
                hoffs = (raws & 1) * EMBED
                gbase = c * CHUNK + gidx * LANES
                for j in range(LANES):
                    r = gidx * LANES + j
                    hoff = hoffs[j]
                    p = lax.rem(gbase + j, MAXSEQ)
                    poff = p * EMBED
                    xs = []
                    for k in range(VPR):
                        t = rows2[b, r, pl.ds(hoff + k * LANES, LANES)]
                        q = pos_v[pl.ds(poff + k * LANES, LANES)]
                        xs.append(t + q)
                    s = (xs[0] + xs[1]) + (xs[2] + xs[3])
                    ssq = (xs[0] * xs[0] + xs[1] * xs[1]) + \
                          (xs[2] * xs[2] + xs[3] * xs[3])
                    mean = jnp.sum(s) * inv_n
                    var = jnp.sum(ssq) * inv_n - mean * mean
                    rstd = _rsqrt(var + eps)
                    scale = jnp.broadcast_to(rstd, (LANES,))
                    mean_v = jnp.broadcast_to(mean, (LANES,))
                    roff = r * EMBED
                    for k in range(VPR):
                        g = gb_v[pl.ds(k * LANES, LANES)]
                        bb = gb_v[pl.ds(EMBED + k * LANES, LANES)]
                        xh = (xs[k] - mean_v) * scale
                        ost2[b, pl.ds(roff + k * LANES, LANES)] = xh * g + bb

            fire_out(base, b)

            @pl.when(c + 2 < NCHUNK)
            def _():
                fire_gather(c + 2, b)

        fire_gather(0, 0)
        fire_gather(1, 1)

        def outer(g, _):
            process(2 * g, 0)
            process(2 * g + 1, 1)
            return 0

        lax.fori_loop(0, NCHUNK // 2, outer, 0)
        wait_out(wstart + (NCHUNK - 2) * CHUNK, 0)
        wait_out(wstart + (NCHUNK - 1) * CHUNK, 1)

    return sc_embed


_sc_embed = _make_sc_call()


@jax.jit
def _run(ids3, tblp, pos1, gb):
    return _sc_embed(ids3, tblp, pos1, gb)


def kernel(input_ids, token_table, pos_table, gamma, beta):
    ids3 = input_ids.reshape(NW, NCHUNK, CHUNK).astype(jnp.int32)
    tblp = token_table.reshape(VOCAB // 2, PAIR)
    pos1 = pos_table.reshape(MAXSEQ * EMBED)
    gb = jnp.concatenate([gamma, beta])
    out = _run(ids3, tblp, pos1, gb)
    return out.reshape(BATCH, SEQ, EMBED)


# speedup vs baseline: 3.6301x; 3.6301x over previous
import functools

import jax
import jax.numpy as jnp
from jax import lax
from jax.experimental import pallas as pl
from jax.experimental.pallas import tpu as pltpu
from jax.experimental.pallas import tpu_sc as plsc

VOCAB = 1000000
EMBED = 64
MAXSEQ = 200
BATCH = 4096
SEQ = 200

TOTAL_ROWS = BATCH * SEQ
LANES = 16
VPR = EMBED // LANES
CHUNK = 100
HALVES = SEQ // CHUNK

_INFO = plsc.get_sparse_core_info()
NC = _INFO.num_cores
NS = _INFO.num_subcores
NW = NC * NS
SEQ_PER_W = BATCH // NW
NCHUNK = SEQ_PER_W * HALVES


def _rsqrt(x):
    i = lax.bitcast_convert_type(x, jnp.int32)
    i = jnp.int32(0x5F3759DF) - lax.shift_right_logical(i, 1)
    y = lax.bitcast_convert_type(i, jnp.float32)
    for _ in range(3):
        y = y * (jnp.float32(1.5) - jnp.float32(0.5) * x * y * y)
    return y


def _make_sc_call():
    mesh = plsc.VectorSubcoreMesh(core_axis_name="c", subcore_axis_name="s")

    @functools.partial(
        pl.kernel,
        mesh=mesh,
        compiler_params=pltpu.CompilerParams(
            needs_layout_passes=False, use_tc_tiling_on_sc=False),
        out_type=jax.ShapeDtypeStruct((BATCH, SEQ, EMBED), jnp.float32),
        scratch_types=[
            pltpu.VMEM((NCHUNK, CHUNK), jnp.int32),
            pltpu.VMEM((2, CHUNK, EMBED), jnp.float32),
            pltpu.VMEM((2, CHUNK, EMBED), jnp.float32),
            pltpu.VMEM((MAXSEQ * EMBED,), jnp.float32),
            pltpu.VMEM((2 * EMBED,), jnp.float32),
            pltpu.SemaphoreType.DMA,
            pltpu.SemaphoreType.DMA,
            pltpu.SemaphoreType.DMA,
            pltpu.SemaphoreType.DMA,
        ],
    )
    def sc_embed(ids_hbm, table_hbm, pos_hbm, gb_hbm, out_hbm,
                 idx_all, rows2, ost2, pos_v, gb_v,
                 gsem0, gsem1, osem0, osem1):
        wid = lax.axis_index("s") * NC + lax.axis_index("c")
        wseq0 = wid * SEQ_PER_W
        gsems = (gsem0, gsem1)
        osems = (osem0, osem1)

        pltpu.sync_copy(ids_hbm.at[wid], idx_all)
        pltpu.sync_copy(pos_hbm, pos_v)
        pltpu.sync_copy(gb_hbm, gb_v)

        inv_n = jnp.float32(1.0 / EMBED)
        eps = jnp.float32(1e-5)

        def fire_gather(c, b):
            pltpu.async_copy(table_hbm.at[idx_all.at[c]], rows2.at[b],
                             gsems[b])

        def wait_gather(c, b):
            pltpu.make_async_copy(table_hbm.at[idx_all.at[c]], rows2.at[b],
                                  gsems[b]).wait()

        def fire_out(c, b):
            seq = wseq0 + lax.shift_right_logical(c, 1)
            sbase = (c & 1) * CHUNK
            pltpu.async_copy(ost2.at[b], out_hbm.at[seq, pl.ds(sbase, CHUNK)],
                             osems[b])

        def wait_out(c, b):
            seq = wseq0 + lax.shift_right_logical(c, 1)
            sbase = (c & 1) * CHUNK
            pltpu.make_async_copy(ost2.at[b],
                                  out_hbm.at[seq, pl.ds(sbase, CHUNK)],
                                  osems[b]).wait()

        def process(c, b):
            wait_gather(c, b)

            @pl.when(c >= 2)
            def _():
                wait_out(c - 2, b)

            pbase = (c & 1) * CHUNK

            @plsc.parallel_loop(0, CHUNK, unroll=4)
            def _row(r):
                poff = (pbase + r) * EMBED
                xs = []
                for k in range(VPR):
                    t = rows2[b, r, pl.ds(k * LANES, LANES)]
                    q = pos_v[pl.ds(poff + k * LANES, LANES)]
                    xs.append(t + q)
                s = (xs[0] + xs[1]) + (xs[2] + xs[3])
                ssq = (xs[0] * xs[0] + xs[1] * xs[1]) + \
                      (xs[2] * xs[2] + xs[3] * xs[3])
                mean = jnp.sum(s) * inv_n
                var = jnp.sum(ssq) * inv_n - mean * mean
                rstd = _rsqrt(var + eps)
                scale = jnp.broadcast_to(rstd, (LANES,))
                mean_v = jnp.broadcast_to(mean, (LANES,))
                for k in range(VPR):
                    g = gb_v[pl.ds(k * LANES, LANES)]
                    bb = gb_v[pl.ds(EMBED + k * LANES, LANES)]
                    xh = (xs[k] - mean_v) * scale
                    ost2[b, r, pl.ds(k * LANES, LANES)] = xh * g + bb

            fire_out(c, b)

            @pl.when(c + 2 < NCHUNK)
            def _():
                fire_gather(c + 2, b)

        fire_gather(0, 0)
        fire_gather(1, 1)

        def outer(g, _):
            process(2 * g, 0)
            process(2 * g + 1, 1)
            return 0

        lax.fori_loop(0, NCHUNK // 2, outer, 0)
        wait_out(NCHUNK - 2, 0)
        wait_out(NCHUNK - 1, 1)

    return sc_embed


_sc_embed = _make_sc_call()


@jax.jit
def _run(ids3, table, pos1, gb):
    return _sc_embed(ids3, table, pos1, gb)


def kernel(input_ids, token_table, pos_table, gamma, beta):
    ids3 = input_ids.reshape(NW, NCHUNK, CHUNK).astype(jnp.int32)
    pos1 = pos_table.reshape(MAXSEQ * EMBED)
    gb = jnp.concatenate([gamma, beta])
    return _run(ids3, token_table, pos1, gb)


# speedup vs baseline: 5.5187x; 1.5202x over previous
import functools

import jax
import jax.numpy as jnp
from jax import lax
from jax.experimental import pallas as pl
from jax.experimental.pallas import tpu as pltpu
from jax.experimental.pallas import tpu_sc as plsc

VOCAB = 1000000
EMBED = 64
MAXSEQ = 200
BATCH = 4096
SEQ = 200

TOTAL_ROWS = BATCH * SEQ
LANES = 16
VPR = EMBED // LANES
WIDE = 128

_INFO = plsc.get_sparse_core_info()
NC = _INFO.num_cores
NS = _INFO.num_subcores
NW = NC * NS
ROWS_PER_W = TOTAL_ROWS // NW
CHUNK = 128
NCHUNK = ROWS_PER_W // CHUNK

BLK = 128
FULL_BLKS = VOCAB // BLK
BLKS_PER_W = FULL_BLKS // NW
EXTRA_BLKS = FULL_BLKS - NW * BLKS_PER_W
TAIL = VOCAB - FULL_BLKS * BLK
PITCH = 136


def _rsqrt(x):
    i = lax.bitcast_convert_type(x, jnp.int32)
    i = jnp.int32(0x5F3759DF) - lax.shift_right_logical(i, 1)
    y = lax.bitcast_convert_type(i, jnp.float32)
    for _ in range(3):
        y = y * (jnp.float32(1.5) - jnp.float32(0.5) * x * y * y)
    return y


def _make_repack():
    mesh = plsc.VectorSubcoreMesh(core_axis_name="c", subcore_axis_name="s")

    @functools.partial(
        pl.kernel,
        mesh=mesh,
        compiler_params=pltpu.CompilerParams(
            needs_layout_passes=False, use_tc_tiling_on_sc=True),
        out_type=jax.ShapeDtypeStruct((VOCAB, WIDE), jnp.float32),
        scratch_types=[
            pltpu.VMEM((2, EMBED, BLK), jnp.float32),
            pltpu.VMEM((BLK * PITCH,), jnp.float32),
            pltpu.VMEM((2, BLK, WIDE), jnp.float32),
            pltpu.SemaphoreType.DMA,
            pltpu.SemaphoreType.DMA,
            pltpu.SemaphoreType.DMA,
            pltpu.SemaphoreType.DMA,
        ],
    )
    def repack(tbl_t_hbm, tail_hbm, out_hbm, in2, scr, ost2,
               isem0, isem1, osem0, osem1):
        wid = lax.axis_index("s") * NC + lax.axis_index("c")
        isems = (isem0, isem1)
        osems = (osem0, osem1)
        zeros = jnp.zeros((LANES,), jnp.float32)
        iv = lax.iota(jnp.int32, LANES) * PITCH

        @plsc.parallel_loop(0, 2 * BLK * VPR, unroll=8)
        def _z(i):
            sl = lax.shift_right_logical(i, 9)
            t = lax.shift_right_logical(i, 2) & 127
            k = i & 3
            ost2[sl, t, pl.ds(EMBED + k * LANES, LANES)] = zeros

        def fire_in(blk, b):
            pltpu.async_copy(tbl_t_hbm.at[:, pl.ds(blk * BLK, BLK)],
                             in2.at[b], isems[b])

        def wait_in(blk, b):
            pltpu.make_async_copy(tbl_t_hbm.at[:, pl.ds(blk * BLK, BLK)],
                                  in2.at[b], isems[b]).wait()

        def fire_out(blk, b):
            pltpu.async_copy(ost2.at[b], out_hbm.at[pl.ds(blk * BLK, BLK)],
                             osems[b])

        def wait_out(blk, b):
            pltpu.make_async_copy(ost2.at[b],
                                  out_hbm.at[pl.ds(blk * BLK, BLK)],
                                  osems[b]).wait()

        def transpose_block(b, ntok):
            @plsc.parallel_loop(0, EMBED * (ntok // LANES), unroll=4)
            def _sc(i):
                e = lax.shift_right_logical(i, 3) if ntok == BLK else \
                    lax.shift_right_logical(i, 2)
                j = (i & 7) if ntok == BLK else (i & 3)
                v = in2[b, e, pl.ds(j * LANES, LANES)]
                idx = iv + (j * (LANES * PITCH) + e)
                plsc.store_scatter(scr, [idx], v)

            @plsc.parallel_loop(0, ntok * VPR, unroll=8)
            def _cp(i):
                t = lax.shift_right_logical(i, 2)
                k = i & 3
                ost2[b, t, pl.ds(k * LANES, LANES)] = \
                    scr[pl.ds(t * PITCH + k * LANES, LANES)]

        def process(blk, b):
            wait_in(blk, b)

            @pl.when(blk >= wid * BLKS_PER_W + 2)
            def _():
                wait_out(blk - 2, b)

            transpose_block(b, BLK)
            fire_out(blk, b)

            @pl.when(blk + 2 < wid * BLKS_PER_W + BLKS_PER_W)
            def _():
                fire_in(blk + 2, b)

        b0 = wid * BLKS_PER_W
        fire_in(b0, 0)
        fire_in(b0 + 1, 1)

        def outer(g, _):
            process(b0 + 2 * g, 0)
            process(b0 + 2 * g + 1, 1)
            return 0

        lax.fori_loop(0, BLKS_PER_W // 2, outer, 0)
        wait_out(b0 + BLKS_PER_W - 2, 0)
        wait_out(b0 + BLKS_PER_W - 1, 1)

        @pl.when(wid < EXTRA_BLKS)
        def _():
            blk = FULL_BLKS - EXTRA_BLKS + wid
            pltpu.sync_copy(tbl_t_hbm.at[:, pl.ds(blk * BLK, BLK)], in2.at[0])
            transpose_block(0, BLK)
            pltpu.sync_copy(ost2.at[0], out_hbm.at[pl.ds(blk * BLK, BLK)])

        @pl.when(wid == EXTRA_BLKS)
        def _():
            c0 = FULL_BLKS * BLK
            pltpu.sync_copy(tail_hbm, ost2.at[1].at[pl.ds(0, TAIL)])
            pltpu.sync_copy(ost2.at[1].at[pl.ds(0, TAIL)],
                            out_hbm.at[pl.ds(c0, TAIL)])

    return repack


def _make_lookup():
    mesh = plsc.VectorSubcoreMesh(core_axis_name="c", subcore_axis_name="s")

    @functools.partial(
        pl.kernel,
        mesh=mesh,
        compiler_params=pltpu.CompilerParams(
            needs_layout_passes=False, use_tc_tiling_on_sc=True),
        out_type=jax.ShapeDtypeStruct((TOTAL_ROWS, EMBED), jnp.float32),
        scratch_types=[
            pltpu.VMEM((NCHUNK, CHUNK), jnp.int32),
            pltpu.VMEM((2, CHUNK, WIDE), jnp.float32),
            pltpu.VMEM((2, CHUNK, EMBED), jnp.float32),
            pltpu.VMEM((MAXSEQ * EMBED,), jnp.float32),
            pltpu.VMEM((2 * EMBED,), jnp.float32),
            pltpu.SemaphoreType.DMA,
            pltpu.SemaphoreType.DMA,
            pltpu.SemaphoreType.DMA,
            pltpu.SemaphoreType.DMA,
        ],
    )
    def sc_embed(ids_hbm, tblw_hbm, pos_hbm, gb_hbm, out_hbm,
                 idx_all, rows2, ost2, pos_v, gb_v,
                 gsem0, gsem1, osem0, osem1):
        wid = lax.axis_index("s") * NC + lax.axis_index("c")
        wstart = wid * ROWS_PER_W
        gsems = (gsem0, gsem1)
        osems = (osem0, osem1)

        pltpu.sync_copy(ids_hbm.at[wid], idx_all)
        pltpu.sync_copy(pos_hbm, pos_v)
        pltpu.sync_copy(gb_hbm, gb_v)

        inv_n = jnp.float32(1.0 / EMBED)
        eps = jnp.float32(1e-5)

        def fire_gather(c, b):
            pltpu.async_copy(tblw_hbm.at[idx_all.at[c]], rows2.at[b],
                             gsems[b])

        def wait_gather(c, b):
            pltpu.make_async_copy(tblw_hbm.at[idx_all.at[c]], rows2.at[b],
                                  gsems[b]).wait()

        def fire_out(base, b):
            pltpu.async_copy(ost2.at[b], out_hbm.at[pl.ds(base, CHUNK)],
                             osems[b])

        def wait_out(base, b):
            pltpu.make_async_copy(ost2.at[b], out_hbm.at[pl.ds(base, CHUNK)],
                                  osems[b]).wait()

        def process(c, b):
            base = wstart + c * CHUNK
            wait_gather(c, b)

            @pl.when(c >= 2)
            def _():
                wait_out(base, b)

            @plsc.parallel_loop(0, CHUNK, unroll=4)
            def _row(r):
                poff = lax.rem(base + r, MAXSEQ) * EMBED
                xs = []
                for k in range(VPR):
                    t = rows2[b, r, pl.ds(k * LANES, LANES)]
                    q = pos_v[pl.ds(poff + k * LANES, LANES)]
                    xs.append(t + q)
                s = (xs[0] + xs[1]) + (xs[2] + xs[3])
                ssq = (xs[0] * xs[0] + xs[1] * xs[1]) + \
                      (xs[2] * xs[2] + xs[3] * xs[3])
                mean = jnp.sum(s) * inv_n
                var = jnp.sum(ssq) * inv_n - mean * mean
                rstd = _rsqrt(var + eps)
                scale = jnp.broadcast_to(rstd, (LANES,))
                mean_v = jnp.broadcast_to(mean, (LANES,))
                for k in range(VPR):
                    g = gb_v[pl.ds(k * LANES, LANES)]
                    bb = gb_v[pl.ds(EMBED + k * LANES, LANES)]
                    xh = (xs[k] - mean_v) * scale
                    ost2[b, r, pl.ds(k * LANES, LANES)] = xh * g + bb

            fire_out(base, b)

            @pl.when(c + 2 < NCHUNK)
            def _():
                fire_gather(c + 2, b)

        fire_gather(0, 0)
        fire_gather(1, 1)

        def outer(g, _):
            process(2 * g, 0)
            process(2 * g + 1, 1)
            return 0

        lax.fori_loop(0, NCHUNK // 2, outer, 0)
        wait_out(wstart + (NCHUNK - 2) * CHUNK, 0)
        wait_out(wstart + (NCHUNK - 1) * CHUNK, 1)

    return sc_embed


_repack = _make_repack()
_lookup = _make_lookup()


@jax.jit
def _run(ids3, tbl_t, tail_w, pos1, gb):
    tblw = _repack(tbl_t, tail_w)
    return _lookup(ids3, tblw, pos1, gb)


def kernel(input_ids, token_table, pos_table, gamma, beta):
    ids3 = input_ids.reshape(NW, NCHUNK, CHUNK).astype(jnp.int32)
    tail_w = jnp.concatenate(
        [token_table[FULL_BLKS * BLK:],
         jnp.zeros((TAIL, EMBED), jnp.float32)], axis=1)
    pos1 = pos_table.reshape(MAXSEQ * EMBED)
    gb = jnp.concatenate([gamma, beta])
    out = _run(ids3, token_table.T, tail_w, pos1, gb)
    return out.reshape(BATCH, SEQ, EMBED)


# speedup vs baseline: 5.8097x; 1.0527x over previous
import functools

import jax
import jax.numpy as jnp
from jax import lax
from jax.experimental import pallas as pl
from jax.experimental.pallas import tpu as pltpu
from jax.experimental.pallas import tpu_sc as plsc

VOCAB = 1000000
EMBED = 64
MAXSEQ = 200
BATCH = 4096
SEQ = 200

TOTAL_ROWS = BATCH * SEQ
LANES = 16
VPR = EMBED // LANES
WIDE = 128

_INFO = plsc.get_sparse_core_info()
NC = _INFO.num_cores
NS = _INFO.num_subcores
NW = NC * NS
ROWS_PER_W = TOTAL_ROWS // NW
CHUNK = 128
NCHUNK = ROWS_PER_W // CHUNK

BLK = 128
FULL_BLKS = VOCAB // BLK
BLKS_PER_W = FULL_BLKS // NW
EXTRA_BLKS = FULL_BLKS - NW * BLKS_PER_W
TAIL = VOCAB - FULL_BLKS * BLK
PITCH = 137


def _rsqrt(x):
    i = lax.bitcast_convert_type(x, jnp.int32)
    i = jnp.int32(0x5F3759DF) - lax.shift_right_logical(i, 1)
    y = lax.bitcast_convert_type(i, jnp.float32)
    for _ in range(3):
        y = y * (jnp.float32(1.5) - jnp.float32(0.5) * x * y * y)
    return y


def _make_repack():
    mesh = plsc.VectorSubcoreMesh(core_axis_name="c", subcore_axis_name="s")

    @functools.partial(
        pl.kernel,
        mesh=mesh,
        compiler_params=pltpu.CompilerParams(
            needs_layout_passes=False, use_tc_tiling_on_sc=True),
        out_type=jax.ShapeDtypeStruct((VOCAB, WIDE), jnp.float32),
        scratch_types=[
            pltpu.VMEM((2, EMBED, BLK), jnp.float32),
            pltpu.VMEM((BLK * PITCH,), jnp.float32),
            pltpu.VMEM((2, BLK, WIDE), jnp.float32),
            pltpu.SemaphoreType.DMA,
            pltpu.SemaphoreType.DMA,
            pltpu.SemaphoreType.DMA,
            pltpu.SemaphoreType.DMA,
        ],
    )
    def repack(tbl_t_hbm, tail_hbm, out_hbm, in2, scr, ost2,
               isem0, isem1, osem0, osem1):
        wid = lax.axis_index("s") * NC + lax.axis_index("c")
        isems = (isem0, isem1)
        osems = (osem0, osem1)
        zeros = jnp.zeros((LANES,), jnp.float32)
        iv = lax.iota(jnp.int32, LANES) * PITCH

        @plsc.parallel_loop(0, 2 * BLK * VPR, unroll=8)
        def _z(i):
            sl = lax.shift_right_logical(i, 9)
            t = lax.shift_right_logical(i, 2) & 127
            k = i & 3
            ost2[sl, t, pl.ds(EMBED + k * LANES, LANES)] = zeros

        def fire_in(blk, b):
            pltpu.async_copy(tbl_t_hbm.at[:, pl.ds(blk * BLK, BLK)],
                             in2.at[b], isems[b])

        def wait_in(blk, b):
            pltpu.make_async_copy(tbl_t_hbm.at[:, pl.ds(blk * BLK, BLK)],
                                  in2.at[b], isems[b]).wait()

        def fire_out(blk, b):
            pltpu.async_copy(ost2.at[b], out_hbm.at[pl.ds(blk * BLK, BLK)],
                             osems[b])

        def wait_out(blk, b):
            pltpu.make_async_copy(ost2.at[b],
                                  out_hbm.at[pl.ds(blk * BLK, BLK)],
                                  osems[b]).wait()

        def transpose_block(b, ntok):
            @plsc.parallel_loop(0, EMBED * (ntok // LANES), unroll=4)
            def _sc(i):
                e = lax.shift_right_logical(i, 3) if ntok == BLK else \
                    lax.shift_right_logical(i, 2)
                j = (i & 7) if ntok == BLK else (i & 3)
                v = in2[b, e, pl.ds(j * LANES, LANES)]
                idx = iv + (j * (LANES * PITCH) + e)
                plsc.store_scatter(scr, [idx], v)

            @plsc.parallel_loop(0, ntok * VPR, unroll=8)
            def _cp(i):
                t = lax.shift_right_logical(i, 2)
                k = i & 3
                ost2[b, t, pl.ds(k * LANES, LANES)] = \
                    scr[pl.ds(t * PITCH + k * LANES, LANES)]

        def process(blk, b):
            wait_in(blk, b)

            @pl.when(blk >= wid * BLKS_PER_W + 2)
            def _():
                wait_out(blk - 2, b)

            transpose_block(b, BLK)
            fire_out(blk, b)

            @pl.when(blk + 2 < wid * BLKS_PER_W + BLKS_PER_W)
            def _():
                fire_in(blk + 2, b)

        b0 = wid * BLKS_PER_W
        fire_in(b0, 0)
        fire_in(b0 + 1, 1)

        def outer(g, _):
            process(b0 + 2 * g, 0)
            process(b0 + 2 * g + 1, 1)
            return 0

        lax.fori_loop(0, BLKS_PER_W // 2, outer, 0)
        wait_out(b0 + BLKS_PER_W - 2, 0)
        wait_out(b0 + BLKS_PER_W - 1, 1)

        @pl.when(wid < EXTRA_BLKS)
        def _():
            blk = FULL_BLKS - EXTRA_BLKS + wid
            pltpu.sync_copy(tbl_t_hbm.at[:, pl.ds(blk * BLK, BLK)], in2.at[0])
            transpose_block(0, BLK)
            pltpu.sync_copy(ost2.at[0], out_hbm.at[pl.ds(blk * BLK, BLK)])

        @pl.when(wid == EXTRA_BLKS)
        def _():
            c0 = FULL_BLKS * BLK
            pltpu.sync_copy(tail_hbm, ost2.at[1].at[pl.ds(0, TAIL)])
            pltpu.sync_copy(ost2.at[1].at[pl.ds(0, TAIL)],
                            out_hbm.at[pl.ds(c0, TAIL)])

    return repack


def _make_lookup():
    mesh = plsc.VectorSubcoreMesh(core_axis_name="c", subcore_axis_name="s")

    @functools.partial(
        pl.kernel,
        mesh=mesh,
        compiler_params=pltpu.CompilerParams(
            needs_layout_passes=False, use_tc_tiling_on_sc=True),
        out_type=jax.ShapeDtypeStruct((TOTAL_ROWS, EMBED), jnp.float32),
        scratch_types=[
            pltpu.VMEM((NCHUNK, CHUNK), jnp.int32),
            pltpu.VMEM((2, CHUNK, WIDE), jnp.float32),
            pltpu.VMEM((2, CHUNK, EMBED), jnp.float32),
            pltpu.VMEM((MAXSEQ * EMBED,), jnp.float32),
            pltpu.VMEM((2 * EMBED,), jnp.float32),
            pltpu.SemaphoreType.DMA,
            pltpu.SemaphoreType.DMA,
            pltpu.SemaphoreType.DMA,
            pltpu.SemaphoreType.DMA,
        ],
    )
    def sc_embed(ids_hbm, tblw_hbm, pos_hbm, gb_hbm, out_hbm,
                 idx_all, rows2, ost2, pos_v, gb_v,
                 gsem0, gsem1, osem0, osem1):
        wid = lax.axis_index("s") * NC + lax.axis_index("c")
        wstart = wid * ROWS_PER_W
        gsems = (gsem0, gsem1)
        osems = (osem0, osem1)

        pltpu.sync_copy(ids_hbm.at[wid], idx_all)
        pltpu.sync_copy(pos_hbm, pos_v)
        pltpu.sync_copy(gb_hbm, gb_v)

        inv_n = jnp.float32(1.0 / EMBED)
        eps = jnp.float32(1e-5)

        def fire_gather(c, b):
            pltpu.async_copy(tblw_hbm.at[idx_all.at[c]], rows2.at[b],
                             gsems[b])

        def wait_gather(c, b):
            pltpu.make_async_copy(tblw_hbm.at[idx_all.at[c]], rows2.at[b],
                                  gsems[b]).wait()

        def fire_out(base, b):
            pltpu.async_copy(ost2.at[b], out_hbm.at[pl.ds(base, CHUNK)],
                             osems[b])

        def wait_out(base, b):
            pltpu.make_async_copy(ost2.at[b], out_hbm.at[pl.ds(base, CHUNK)],
                                  osems[b]).wait()

        def process(c, b):
            base = wstart + c * CHUNK
            wait_gather(c, b)

            @pl.when(c >= 2)
            def _():
                wait_out(base, b)

            @plsc.parallel_loop(0, CHUNK, unroll=4)
            def _row(r):
                poff = lax.rem(base + r, MAXSEQ) * EMBED
                xs = []
                for k in range(VPR):
                    t = rows2[b, r, pl.ds(k * LANES, LANES)]
                    q = pos_v[pl.ds(poff + k * LANES, LANES)]
                    xs.append(t + q)
                s = (xs[0] + xs[1]) + (xs[2] + xs[3])
                ssq = (xs[0] * xs[0] + xs[1] * xs[1]) + \
                      (xs[2] * xs[2] + xs[3] * xs[3])
                mean = jnp.sum(s) * inv_n
                var = jnp.sum(ssq) * inv_n - mean * mean
                rstd = _rsqrt(var + eps)
                scale = jnp.broadcast_to(rstd, (LANES,))
                mean_v = jnp.broadcast_to(mean, (LANES,))
                for k in range(VPR):
                    g = gb_v[pl.ds(k * LANES, LANES)]
                    bb = gb_v[pl.ds(EMBED + k * LANES, LANES)]
                    xh = (xs[k] - mean_v) * scale
                    ost2[b, r, pl.ds(k * LANES, LANES)] = xh * g + bb

            fire_out(base, b)

            @pl.when(c + 2 < NCHUNK)
            def _():
                fire_gather(c + 2, b)

        fire_gather(0, 0)
        fire_gather(1, 1)

        def outer(g, _):
            process(2 * g, 0)
            process(2 * g + 1, 1)
            return 0

        lax.fori_loop(0, NCHUNK // 2, outer, 0)
        wait_out(wstart + (NCHUNK - 2) * CHUNK, 0)
        wait_out(wstart + (NCHUNK - 1) * CHUNK, 1)

    return sc_embed


_repack = _make_repack()
_lookup = _make_lookup()


@jax.jit
def _run(ids3, tbl_t, tail_w, pos1, gb):
    tblw = _repack(tbl_t, tail_w)
    return _lookup(ids3, tblw, pos1, gb)


def kernel(input_ids, token_table, pos_table, gamma, beta):
    ids3 = input_ids.reshape(NW, NCHUNK, CHUNK).astype(jnp.int32)
    tail_w = jnp.concatenate(
        [token_table[FULL_BLKS * BLK:],
         jnp.zeros((TAIL, EMBED), jnp.float32)], axis=1)
    pos1 = pos_table.reshape(MAXSEQ * EMBED)
    gb = jnp.concatenate([gamma, beta])
    out = _run(ids3, token_table.T, tail_w, pos1, gb)
    return out.reshape(BATCH, SEQ, EMBED)
